# Initial kernel scaffold; baseline (speedup 1.0000x reference)
import jax
import jax.numpy as jnp
from jax.experimental import pallas as pl


def kernel(xfeat, T, edge_index, edge_feat, Tij_hat, u, batch, W1a, b1a, W1b, b1b, W2a, b2a, W2b, b2b):
    raise NotImplementedError("write your pallas kernel here")


# speedup vs baseline: 7.4538x; 7.4538x over previous
import functools
import math

import jax
import jax.numpy as jnp
from jax import lax
from jax.experimental import pallas as pl
from jax.experimental.pallas import tpu as pltpu
from jax.experimental.pallas import tpu_sc as plsc

DIM = 12
DOF = 6
NC = 2
NS = 16
NW = NC * NS
EBLK = 128


def _cols(tm, n):
    return [tm[:, k:k + 1] for k in range(n)]


def _unpack_pose(cols12):
    R = [[cols12[4 * r + c] for c in range(3)] for r in range(3)]
    t = [cols12[4 * r + 3] for r in range(3)]
    return R, t


def _pack_pose(R, t):
    out = []
    for r in range(3):
        out.extend([R[r][0], R[r][1], R[r][2], t[r]])
    return out


def _mat3(A, B):
    return [[A[r][0] * B[0][c] + A[r][1] * B[1][c] + A[r][2] * B[2][c]
             for c in range(3)] for r in range(3)]


def _mv3(A, v):
    return [A[r][0] * v[0] + A[r][1] * v[1] + A[r][2] * v[2] for r in range(3)]


def _t3(A):
    return [[A[c][r] for c in range(3)] for r in range(3)]


def _neg3(v):
    return [-x for x in v]


def _add3(a, b):
    return [x + y for x, y in zip(a, b)]


def _k1_body(x_ref, wa_ref, wb_ref, g_ref):
    x = x_ref[...]
    g_ref[...] = jnp.concatenate(
        [jnp.dot(x, wa_ref[...], preferred_element_type=jnp.float32),
         jnp.dot(x, wb_ref[...], preferred_element_type=jnp.float32)], axis=1)


def _k1(xfeat, wa, wb, blk=2000):
    n, nd = xfeat.shape
    h = wa.shape[1]
    grid = n // blk
    return pl.pallas_call(
        _k1_body,
        grid=(grid,),
        in_specs=[
            pl.BlockSpec((blk, nd), lambda b: (b, 0)),
            pl.BlockSpec((nd, h), lambda b: (0, 0)),
            pl.BlockSpec((nd, h), lambda b: (0, 0)),
        ],
        out_specs=pl.BlockSpec((blk, 2 * h), lambda b: (b, 0)),
        out_shape=jax.ShapeDtypeStruct((n, 2 * h), jnp.float32),
    )(xfeat, wa, wb)


K2BLK = 2 * EBLK


def _k2_body(g, t16, ei, s_out, ti_out, tj_out,
             idx_i0, idx_i1, idx_j0, idx_j1,
             bufi, bufj, sbij, sbji, bufti, buftj, sems):
    nblk_total = ei.shape[1] // K2BLK
    h = sbij.shape[1]
    wid = lax.axis_index("s") * NC + lax.axis_index("c")
    nblk = (nblk_total - wid + NW - 1) // NW
    e_total = ei.shape[1]

    def block_body(t, carry):
        blk = wid + t * NW
        e0 = blk * K2BLK
        ci0 = pltpu.async_copy(ei.at[0, pl.ds(e0, EBLK)], idx_i0, sems.at[0])
        ci1 = pltpu.async_copy(ei.at[0, pl.ds(e0 + EBLK, EBLK)], idx_i1, sems.at[1])
        cj0 = pltpu.async_copy(ei.at[1, pl.ds(e0, EBLK)], idx_j0, sems.at[2])
        cj1 = pltpu.async_copy(ei.at[1, pl.ds(e0 + EBLK, EBLK)], idx_j1, sems.at[3])
        ci0.wait()
        ci1.wait()
        cj0.wait()
        cj1.wait()
        cps = [
            pltpu.async_copy(g.at[idx_i0], bufi.at[pl.ds(0, EBLK)], sems.at[0]),
            pltpu.async_copy(g.at[idx_i1], bufi.at[pl.ds(EBLK, EBLK)], sems.at[1]),
            pltpu.async_copy(g.at[idx_j0], bufj.at[pl.ds(0, EBLK)], sems.at[2]),
            pltpu.async_copy(g.at[idx_j1], bufj.at[pl.ds(EBLK, EBLK)], sems.at[3]),
            pltpu.async_copy(t16.at[idx_i0], bufti.at[pl.ds(0, EBLK)], sems.at[4]),
            pltpu.async_copy(t16.at[idx_i1], bufti.at[pl.ds(EBLK, EBLK)], sems.at[5]),
            pltpu.async_copy(t16.at[idx_j0], buftj.at[pl.ds(0, EBLK)], sems.at[6]),
            pltpu.async_copy(t16.at[idx_j1], buftj.at[pl.ds(EBLK, EBLK)], sems.at[7]),
        ]
        for cp in cps[:4]:
            cp.wait()

        def addrow(r, c2):
            for c in range(h // 16):
                sbij[r, pl.ds(c * 16, 16)] = (
                    bufi[r, pl.ds(c * 16, 16)] + bufj[r, pl.ds(h + c * 16, 16)])
                sbji[r, pl.ds(c * 16, 16)] = (
                    bufj[r, pl.ds(c * 16, 16)] + bufi[r, pl.ds(h + c * 16, 16)])
            return c2

        lax.fori_loop(0, K2BLK, addrow, 0)
        pltpu.sync_copy(sbij, s_out.at[pl.ds(e0, K2BLK)])
        pltpu.sync_copy(sbji, s_out.at[pl.ds(e_total + e0, K2BLK)])
        for cp in cps[4:]:
            cp.wait()
        pltpu.sync_copy(bufti, ti_out.at[pl.ds(e0, K2BLK)])
        pltpu.sync_copy(buftj, tj_out.at[pl.ds(e0, K2BLK)])
        return carry

    lax.fori_loop(0, nblk, block_body, 0)


def _k2(g, t16, ei):
    n, h2 = g.shape
    h = h2 // 2
    e = ei.shape[1]
    mesh = plsc.VectorSubcoreMesh(core_axis_name="c", subcore_axis_name="s")
    kfn = pl.kernel(
        _k2_body,
        out_type=(
            jax.ShapeDtypeStruct((2 * e, h), jnp.float32),
            jax.ShapeDtypeStruct((e, 16), jnp.float32),
            jax.ShapeDtypeStruct((e, 16), jnp.float32),
        ),
        mesh=mesh,
        scratch_types=[
            pltpu.VMEM((EBLK,), jnp.int32),
            pltpu.VMEM((EBLK,), jnp.int32),
            pltpu.VMEM((EBLK,), jnp.int32),
            pltpu.VMEM((EBLK,), jnp.int32),
            pltpu.VMEM((K2BLK, h2), jnp.float32),
            pltpu.VMEM((K2BLK, h2), jnp.float32),
            pltpu.VMEM((K2BLK, h), jnp.float32),
            pltpu.VMEM((K2BLK, h), jnp.float32),
            pltpu.VMEM((K2BLK, 16), jnp.float32),
            pltpu.VMEM((K2BLK, 16), jnp.float32),
            pltpu.SemaphoreType.DMA((8,)),
        ],
        compiler_params=pltpu.CompilerParams(use_tc_tiling_on_sc=False),
    )
    return kfn(g, t16, ei)


def _rows(tm, n):
    return [tm[k:k + 1, :] for k in range(n)]


def _k3_body(s_ref, ti_ref, tj_ref, th_ref, ef_ref,
             wef_ref, wt_ref, b1a_ref, w1b_ref, b1b_ref, f_ref):
    d = pl.program_id(0)
    is_ij = (d == 0)
    ti = _rows(ti_ref[...], 12)
    tj = _rows(tj_ref[...], 12)
    th = _rows(th_ref[...], 12)

    def sel(a, b):
        return jnp.where(is_ij, a, b)

    p = [sel(a, b) for a, b in zip(tj, ti)]
    q = [sel(a, b) for a, b in zip(ti, tj)]
    Rp, tp = _unpack_pose(p)
    Rq, tq = _unpack_pose(q)
    Rm = _mat3(Rp, _t3(Rq))
    tm = _add3(tp, _neg3(_mv3(Rm, tq)))
    Rh, tth = _unpack_pose(th)
    Rht = _t3(Rh)
    nh = _neg3(_mv3(Rht, tth))
    Hr = [[sel(Rht[r][c], Rh[r][c]) for c in range(3)] for r in range(3)]
    Ht = [sel(nh[r], tth[r]) for r in range(3)]
    Re = _mat3(Rm, Hr)
    te = _add3(_mv3(Rm, Ht), tm)
    terr_t = jnp.concatenate(_pack_pose(Re, te), axis=0)

    pre = (s_ref[0]
           + jnp.dot(ef_ref[...], wef_ref[...], preferred_element_type=jnp.float32)
           + lax.dot_general(terr_t, wt_ref[...], (((0,), (0,)), ((), ())),
                             preferred_element_type=jnp.float32)
           + b1a_ref[...])
    h1 = jnp.maximum(pre, 0.0)
    h2 = jnp.dot(h1, w1b_ref[...], preferred_element_type=jnp.float32) + b1b_ref[...]
    f_ref[0] = jnp.maximum(h2, 0.0)


def _k3(s3, ti_t, tj_t, th_t, ef, wef, wt, b1a, w1b, b1b, blk=640):
    e = ef.shape[0]
    h = wef.shape[1]
    nb = e // blk
    return pl.pallas_call(
        _k3_body,
        grid=(2, nb),
        in_specs=[
            pl.BlockSpec((1, blk, h), lambda d, b: (d, b, 0)),
            pl.BlockSpec((16, blk), lambda d, b: (0, b)),
            pl.BlockSpec((16, blk), lambda d, b: (0, b)),
            pl.BlockSpec((12, blk), lambda d, b: (0, b)),
            pl.BlockSpec((blk, 16), lambda d, b: (b, 0)),
            pl.BlockSpec((16, h), lambda d, b: (0, 0)),
            pl.BlockSpec((12, h), lambda d, b: (0, 0)),
            pl.BlockSpec((1, h), lambda d, b: (0, 0)),
            pl.BlockSpec((h, h), lambda d, b: (0, 0)),
            pl.BlockSpec((1, h), lambda d, b: (0, 0)),
        ],
        out_specs=pl.BlockSpec((1, blk, h), lambda d, b: (d, b, 0)),
        out_shape=jax.ShapeDtypeStruct((2, e, h), jnp.float32),
    )(s3, ti_t, tj_t, th_t, ef, wef, wt, b1a, w1b, b1b)


def _k4_body(f, dstv, z64, z16, p_out, c_out,
             idx_v, fbuf, ones_v, acc, cnt, sems):
    rows = f.shape[0]
    nblk_total = rows // EBLK
    n = z64.shape[0]
    cid = lax.axis_index("c")
    sid = lax.axis_index("s")
    wid = sid * NC + cid
    rows_per_sc = n // NS

    @pl.when(sid == 0)
    def _init():
        pltpu.sync_copy(z64, acc)
        pltpu.sync_copy(z16, cnt)

    def fill_ones(r, c2):
        ones_v[r, pl.ds(0, 16)] = jnp.ones((16,), jnp.float32)
        return c2

    lax.fori_loop(0, EBLK, fill_ones, 0)
    plsc.subcore_barrier()

    nblk = (nblk_total - wid + NW - 1) // NW

    def block_body(t, carry):
        blk = wid + t * NW
        r0 = blk * EBLK
        pltpu.sync_copy(dstv.at[pl.ds(r0, EBLK)], idx_v)
        pltpu.sync_copy(f.at[pl.ds(r0, EBLK)], fbuf)
        pltpu.sync_copy(fbuf, acc.at[idx_v], add=True)
        pltpu.sync_copy(ones_v, cnt.at[idx_v], add=True)
        return carry

    lax.fori_loop(0, nblk, block_body, 0)
    plsc.subcore_barrier()
    pltpu.sync_copy(acc.at[pl.ds(sid * rows_per_sc, rows_per_sc)],
                    p_out.at[cid, pl.ds(sid * rows_per_sc, rows_per_sc)])
    pltpu.sync_copy(cnt.at[pl.ds(sid * rows_per_sc, rows_per_sc)],
                    c_out.at[cid, pl.ds(sid * rows_per_sc, rows_per_sc)])


def _k4(fflat, dstv, z64, z16):
    rows, h = fflat.shape
    n = z64.shape[0]
    mesh = plsc.VectorSubcoreMesh(core_axis_name="c", subcore_axis_name="s")
    kfn = pl.kernel(
        _k4_body,
        out_type=(
            jax.ShapeDtypeStruct((NC, n, h), jnp.float32),
            jax.ShapeDtypeStruct((NC, n, 16), jnp.float32),
        ),
        mesh=mesh,
        scratch_types=[
            pltpu.VMEM((EBLK,), jnp.int32),
            pltpu.VMEM((EBLK, h), jnp.float32),
            pltpu.VMEM((EBLK, 16), jnp.float32),
            pltpu.VMEM_SHARED((n, h), jnp.float32),
            pltpu.VMEM_SHARED((n, 16), jnp.float32),
            pltpu.SemaphoreType.DMA((2,)),
        ],
        compiler_params=pltpu.CompilerParams(use_tc_tiling_on_sc=False),
    )
    return kfn(fflat, dstv, z64, z16)


def _k5_body(p_ref, c_ref, x_ref, t_ref, u_ref, b_ref,
             w2a1_ref, w2a2_ref, w2a3_ref, b2a_ref, w2b_ref, b2b_ref,
             xo_ref, taux_ref):
    sums = p_ref[0] + p_ref[1]
    cnt = c_ref[0][:, 0:1] + c_ref[1][:, 0:1]
    aggr = sums / jnp.maximum(cnt, 1.0)
    x = x_ref[...]
    rows = x.shape[0]
    bvals = b_ref[...]
    nb = u_ref[...].shape[0]
    onehot = (lax.broadcasted_iota(jnp.int32, (rows, nb), 1) == bvals
              ).astype(jnp.float32)
    v = jnp.dot(u_ref[...], w2a3_ref[...], preferred_element_type=jnp.float32)
    h = (jnp.dot(aggr, w2a1_ref[...], preferred_element_type=jnp.float32)
         + jnp.dot(x, w2a2_ref[...], preferred_element_type=jnp.float32)
         + jnp.dot(onehot, v, preferred_element_type=jnp.float32)
         + b2a_ref[...])
    h = jnp.maximum(h, 0.0)
    out = jnp.dot(h, w2b_ref[...], preferred_element_type=jnp.float32) + b2b_ref[...]
    xo_ref[...] = x + out[:, :-DOF]

    xi_t = jnp.transpose(out[:, -DOF:])
    rho = _rows(xi_t, 6)[:3]
    phi = _rows(xi_t, 6)[3:]
    th2_raw = phi[0] * phi[0] + phi[1] * phi[1] + phi[2] * phi[2]
    th_raw = jnp.sqrt(th2_raw)
    scale = jnp.pi * jnp.tanh(th_raw / jnp.pi) / (th_raw + 1e-8)
    ph = [c * scale for c in phi]
    theta = jnp.sqrt(ph[0] * ph[0] + ph[1] * ph[1] + ph[2] * ph[2])

    th2 = ph[0] * ph[0] + ph[1] * ph[1] + ph[2] * ph[2]
    th = jnp.sqrt(th2)
    small = th < 1e-8
    th_s = jnp.where(small, 1.0, th)
    A = jnp.where(small, 1.0 - th2 / 6.0, jnp.sin(th_s) / th_s)
    Bc = jnp.where(small, 0.5 - th2 / 24.0,
                   (1.0 - jnp.cos(th_s)) / (th_s * th_s))
    Cc = jnp.where(small, 1.0 / 6.0 - th2 / 120.0,
                   (th_s - jnp.sin(th_s)) / (th_s * th_s * th_s))
    zero = jnp.zeros_like(ph[0])
    K = [[zero, -ph[2], ph[1]],
         [ph[2], zero, -ph[0]],
         [-ph[1], ph[0], zero]]
    K2 = _mat3(K, K)
    eye = [[(1.0 if r == c else 0.0) + zero for c in range(3)] for r in range(3)]
    R = [[eye[r][c] + A * K[r][c] + Bc * K2[r][c] for c in range(3)]
         for r in range(3)]
    V = [[eye[r][c] + Bc * K[r][c] + Cc * K2[r][c] for c in range(3)]
         for r in range(3)]
    te = _mv3(V, rho)
    tn = _rows(t_ref[...], 12)
    Rn, ttn = _unpack_pose(tn)
    Ro = _mat3(R, Rn)
    to = _add3(_mv3(R, ttn), te)
    packed = _pack_pose(Ro, to) + [theta, zero, zero, zero]
    taux_ref[...] = jnp.concatenate(packed, axis=0)


def _k5(p, c, xfeat, t12, u, b2d, w2a1, w2a2, w2a3, b2a, w2b, b2b, blk=1024):
    n, nd = xfeat.shape
    h = w2a1.shape[1]
    out2 = w2b.shape[1]
    nb = u.shape[0]
    grid = n // blk
    return pl.pallas_call(
        _k5_body,
        grid=(grid,),
        in_specs=[
            pl.BlockSpec((NC, blk, h), lambda b: (0, b, 0)),
            pl.BlockSpec((NC, blk, 16), lambda b: (0, b, 0)),
            pl.BlockSpec((blk, nd), lambda b: (b, 0)),
            pl.BlockSpec((12, blk), lambda b: (0, b)),
            pl.BlockSpec((nb, u.shape[1]), lambda b: (0, 0)),
            pl.BlockSpec((blk, 1), lambda b: (b, 0)),
            pl.BlockSpec((h, h), lambda b: (0, 0)),
            pl.BlockSpec((nd, h), lambda b: (0, 0)),
            pl.BlockSpec((u.shape[1], h), lambda b: (0, 0)),
            pl.BlockSpec((1, h), lambda b: (0, 0)),
            pl.BlockSpec((h, out2), lambda b: (0, 0)),
            pl.BlockSpec((1, out2), lambda b: (0, 0)),
        ],
        out_specs=[
            pl.BlockSpec((blk, nd), lambda b: (b, 0)),
            pl.BlockSpec((16, blk), lambda b: (0, b)),
        ],
        out_shape=[
            jax.ShapeDtypeStruct((n, nd), jnp.float32),
            jax.ShapeDtypeStruct((16, n), jnp.float32),
        ],
    )(p, c, xfeat, t12, u, b2d, w2a1, w2a2, w2a3, b2a, w2b, b2b)


def kernel(xfeat, T, edge_index, edge_feat, Tij_hat, u, batch,
           W1a, b1a, W1b, b1b, W2a, b2a, W2b, b2b):
    n, nd = xfeat.shape
    e = edge_index.shape[1]
    h = W1b.shape[0]

    t12 = T.reshape(n, DIM)
    t16 = jnp.pad(t12, ((0, 0), (0, 4)))
    th12 = Tij_hat.reshape(e, DIM)
    ei = edge_index.astype(jnp.int32)
    dstv = jnp.concatenate([ei[1], ei[0]])
    z64 = jnp.zeros((n, h), jnp.float32)
    z16 = jnp.zeros((n, 16), jnp.float32)

    g = _k1(xfeat, W1a[:nd], W1a[nd:2 * nd])
    s, ti, tj = _k2(g, t16, ei)
    f = _k3(s.reshape(2, e, h), ti.T, tj.T, th12.T, edge_feat,
            W1a[2 * nd:2 * nd + 16], W1a[2 * nd + 16:], b1a.reshape(1, h),
            W1b, b1b.reshape(1, h))
    p, c = _k4(f.reshape(2 * e, h), dstv, z64, z16)
    np_ = ((n + 1023) // 1024) * 1024
    pad = np_ - n
    xf_out, taux_t = _k5(jnp.pad(p, ((0, 0), (0, pad), (0, 0))),
                         jnp.pad(c, ((0, 0), (0, pad), (0, 0))),
                         jnp.pad(xfeat, ((0, pad), (0, 0))),
                         jnp.pad(t12.T, ((0, 0), (0, pad))), u,
                         jnp.pad(batch.astype(jnp.int32).reshape(n, 1),
                                 ((0, pad), (0, 0))),
                         W2a[:h], W2a[h:h + nd], W2a[h + nd:],
                         b2a.reshape(1, h), W2b, b2b.reshape(1, W2b.shape[1]))
    t_out = taux_t[:DIM, :n].T.reshape(n, 3, 4)
    theta = taux_t[DIM, :n]
    return (xf_out[:n], t_out, theta)
